# baseline (speedup 1.0000x reference)
import functools
import math

import jax
import jax.numpy as jnp
import numpy as np
from jax import lax
from jax.experimental import pallas as pl
from jax.experimental.pallas import tpu as pltpu
from jax.experimental.pallas import tpu_sc as plsc

_B = 1024
_D = 64
_K = 100000
_TEMP = 0.05
_CHUNK = 2048
_NB = -(-_K // _CHUNK)
_REM = _K - (_NB - 1) * _CHUNK
_NG = 2 * _B
_LOG2E = math.log2(math.e)
_LN2 = math.log(2.0)


def _gather_rows(table, idx):
    info = plsc.get_sparse_core_info()
    nc = info.num_cores
    nw = nc * info.num_subcores
    n = idx.shape[0]
    b_per_w = n // nw

    @functools.partial(
        pl.kernel,
        mesh=plsc.VectorSubcoreMesh(core_axis_name="c", subcore_axis_name="s"),
        out_type=jax.ShapeDtypeStruct((n, 2 * _D), jnp.float32),
        scratch_types=[
            pltpu.VMEM((b_per_w,), jnp.int32),
            pltpu.VMEM((b_per_w, 2 * _D), jnp.float32),
            pltpu.SemaphoreType.DMA,
        ],
    )
    def gath(table_hbm, idx_hbm, out_hbm, idx_v, rows_v, sem):
        wid = lax.axis_index("s") * nc + lax.axis_index("c")
        base = wid * b_per_w
        pltpu.sync_copy(idx_hbm.at[pl.ds(base, b_per_w)], idx_v)
        pltpu.async_copy(table_hbm.at[idx_v], rows_v, sem).wait()
        pltpu.sync_copy(rows_v, out_hbm.at[pl.ds(base, b_per_w)])

    return gath(table, idx)


def _lse_kernel(x_ref, memt_ref, lse_ref, xs_ref, m_ref, s_ref):
    i = pl.program_id(0)

    @pl.when(i == 0)
    def _init():
        x = x_ref[...]
        nrm = jnp.sqrt(jnp.sum(x * x, axis=1, keepdims=True))
        xs_ref[...] = x * (_LOG2E / _TEMP) / jnp.maximum(nrm, 1e-12)
        m_ref[...] = jnp.full((_B, 1), -1e30, jnp.float32)
        s_ref[...] = jnp.zeros((_B, 1), jnp.float32)

    dot = lax.dot_general(xs_ref[...], memt_ref[...],
                          (((1,), (0,)), ((), ())),
                          preferred_element_type=jnp.float32)

    def _update(dotv):
        cm = jnp.max(dotv, axis=1, keepdims=True)
        m_old = m_ref[...]
        m_new = jnp.maximum(m_old, cm)
        e = jnp.exp2(dotv - m_new)
        ssum = jnp.sum(e, axis=1, keepdims=True)
        s_ref[...] = s_ref[...] * jnp.exp2(m_old - m_new) + ssum
        m_ref[...] = m_new

    @pl.when(i < _NB - 1)
    def _full():
        _update(dot)

    @pl.when(i == _NB - 1)
    def _partial():
        col = lax.broadcasted_iota(jnp.int32, (_B, _CHUNK), 1)
        _update(jnp.where(col < _REM, dot, -1e30))

    @pl.when(i == _NB - 1)
    def _fin():
        lse_ref[...] = m_ref[...] * _LN2 + jnp.log(s_ref[...])


def _combine_kernel(x_ref, b_ref, g_ref, par_ref, lse_ref, out_ref):
    x = x_ref[...]
    xn = x / jnp.maximum(
        jnp.sqrt(jnp.sum(x * x, axis=1, keepdims=True)), 1e-12)
    par = par_ref[...]
    sel = jnp.where(par == 0, g_ref[:, 0:_D], g_ref[:, _D:2 * _D])
    g1 = sel[0:_B, :]
    g2 = sel[_B:_NG, :]
    tdot = jnp.sum(xn * g1, axis=1, keepdims=True) / _TEMP
    loss_ce = jnp.mean(lse_ref[...] - tdot)
    b = b_ref[...]
    bn = b / jnp.maximum(
        jnp.sqrt(jnp.sum(b * b, axis=1, keepdims=True)), 1e-12)
    diff = bn - g2
    loss_d = (0.007 / 0.3) * jnp.sqrt(jnp.sum(diff * diff))
    out_ref[0, 0] = loss_ce + loss_d


def _tc_lse(x, memt):
    return pl.pallas_call(
        _lse_kernel,
        grid=(_NB,),
        in_specs=[
            pl.BlockSpec((_B, _D), lambda i: (0, 0)),
            pl.BlockSpec((_D, _CHUNK), lambda i: (0, i)),
        ],
        out_specs=pl.BlockSpec((_B, 1), lambda i: (0, 0)),
        out_shape=jax.ShapeDtypeStruct((_B, 1), jnp.float32),
        scratch_shapes=[
            pltpu.VMEM((_B, _D), jnp.float32),
            pltpu.VMEM((_B, 1), jnp.float32),
            pltpu.VMEM((_B, 1), jnp.float32),
        ],
    )(x, memt)


def _tc_combine(x, b, g, par, lse):
    return pl.pallas_call(
        _combine_kernel,
        in_specs=[
            pl.BlockSpec((_B, _D), lambda: (0, 0)),
            pl.BlockSpec((_B, _D), lambda: (0, 0)),
            pl.BlockSpec((_NG, 2 * _D), lambda: (0, 0)),
            pl.BlockSpec((_NG, 1), lambda: (0, 0)),
            pl.BlockSpec((_B, 1), lambda: (0, 0)),
        ],
        out_specs=pl.BlockSpec(memory_space=pltpu.SMEM),
        out_shape=jax.ShapeDtypeStruct((1, 1), jnp.float32),
    )(x, b, g, par, lse)


_K2 = _K // 2


def kernel(backbone_inputs, inputs, targets, memory_features):
    idx = jnp.concatenate([targets, jnp.repeat(targets[: _B // 4], 4)])
    mem2 = jnp.reshape(memory_features, (_K2, 2 * _D))
    g = _gather_rows(mem2, idx // 2)
    par = (idx % 2).astype(jnp.int32)[:, None]
    lse = _tc_lse(inputs, memory_features.T)
    loss = _tc_combine(inputs, backbone_inputs, g, par, lse)
    return loss[0, 0]


# speedup vs baseline: 1.4231x; 1.4231x over previous
import functools
import math

import jax
import jax.numpy as jnp
import numpy as np
from jax import lax
from jax.experimental import pallas as pl
from jax.experimental.pallas import tpu as pltpu
from jax.experimental.pallas import tpu_sc as plsc

_B = 1024
_D = 64
_K = 100000
_TEMP = 0.05
_CHUNK = 4096
_NB = -(-_K // _CHUNK)
_REM = _K - (_NB - 1) * _CHUNK
_NG = 2 * _B
_LOG2E = math.log2(math.e)
_LN2 = math.log(2.0)


def _gather_rows(table, idx):
    info = plsc.get_sparse_core_info()
    nc = info.num_cores
    nw = nc * info.num_subcores
    n = idx.shape[0]
    b_per_w = n // nw

    @functools.partial(
        pl.kernel,
        mesh=plsc.VectorSubcoreMesh(core_axis_name="c", subcore_axis_name="s"),
        out_type=jax.ShapeDtypeStruct((n, 2 * _D), jnp.float32),
        scratch_types=[
            pltpu.VMEM((b_per_w,), jnp.int32),
            pltpu.VMEM((b_per_w, 2 * _D), jnp.float32),
            pltpu.SemaphoreType.DMA,
        ],
    )
    def gath(table_hbm, idx_hbm, out_hbm, idx_v, rows_v, sem):
        wid = lax.axis_index("s") * nc + lax.axis_index("c")
        base = wid * b_per_w
        pltpu.sync_copy(idx_hbm.at[pl.ds(base, b_per_w)], idx_v)
        pltpu.async_copy(table_hbm.at[idx_v], rows_v, sem).wait()
        pltpu.sync_copy(rows_v, out_hbm.at[pl.ds(base, b_per_w)])

    return gath(table, idx)


def _lse_kernel(x_ref, memt_ref, eye_ref, lse_ref, pad_ref,
                xs_ref, m_ref, s_ref):
    i = pl.program_id(0)

    @pl.when(i == 0)
    def _init():
        x = x_ref[...]
        nrm = jnp.sqrt(jnp.sum(x * x, axis=1, keepdims=True))
        xs_ref[...] = x * (_LOG2E / _TEMP) / jnp.maximum(nrm, 1e-12)
        m_ref[...] = jnp.full((_B, 1), -1e30, jnp.float32)
        s_ref[...] = jnp.zeros((_B, 1), jnp.float32)

    memt = memt_ref[...]
    pad_ref[:, 0:_D] = jnp.transpose(memt)
    dot = lax.dot_general(xs_ref[...], memt, (((1,), (0,)), ((), ())),
                          preferred_element_type=jnp.float32)

    def _update(dotv):
        cm = jnp.max(dotv, axis=1, keepdims=True)
        m_old = m_ref[...]
        m_new = jnp.maximum(m_old, cm)
        e = jnp.exp2(dotv - m_new)
        ssum = jnp.sum(e, axis=1, keepdims=True)
        s_ref[...] = s_ref[...] * jnp.exp2(m_old - m_new) + ssum
        m_ref[...] = m_new

    @pl.when(i < _NB - 1)
    def _full():
        _update(dot)

    @pl.when(i == _NB - 1)
    def _partial():
        col = lax.broadcasted_iota(jnp.int32, (_B, _CHUNK), 1)
        _update(jnp.where(col < _REM, dot, -1e30))

    @pl.when(i == _NB - 1)
    def _fin():
        lse_ref[...] = m_ref[...] * _LN2 + jnp.log(s_ref[...])


def _combine_kernel(x_ref, b_ref, g_ref, lse_ref, out_ref):
    x = x_ref[...]
    xn = x / jnp.maximum(
        jnp.sqrt(jnp.sum(x * x, axis=1, keepdims=True)), 1e-12)
    g1 = g_ref[0:_B, 0:_D]
    g2 = g_ref[_B:_NG, 0:_D]
    tdot = jnp.sum(xn * g1, axis=1, keepdims=True) / _TEMP
    loss_ce = jnp.mean(lse_ref[...] - tdot)
    b = b_ref[...]
    bn = b / jnp.maximum(
        jnp.sqrt(jnp.sum(b * b, axis=1, keepdims=True)), 1e-12)
    diff = bn - g2
    loss_d = (0.007 / 0.3) * jnp.sqrt(jnp.sum(diff * diff))
    out_ref[0, 0] = loss_ce + loss_d


_EYE = np.eye(_D, 2 * _D, dtype=np.float32)


def _tc_lse(x, memt):
    return pl.pallas_call(
        _lse_kernel,
        grid=(_NB,),
        in_specs=[
            pl.BlockSpec((_B, _D), lambda i: (0, 0)),
            pl.BlockSpec((_D, _CHUNK), lambda i: (0, i)),
            pl.BlockSpec((_D, 2 * _D), lambda i: (0, 0)),
        ],
        out_specs=[
            pl.BlockSpec((_B, 1), lambda i: (0, 0)),
            pl.BlockSpec((_CHUNK, 2 * _D), lambda i: (i, 0)),
        ],
        out_shape=[
            jax.ShapeDtypeStruct((_B, 1), jnp.float32),
            jax.ShapeDtypeStruct((_K, 2 * _D), jnp.float32),
        ],
        scratch_shapes=[
            pltpu.VMEM((_B, _D), jnp.float32),
            pltpu.VMEM((_B, 1), jnp.float32),
            pltpu.VMEM((_B, 1), jnp.float32),
        ],
        compiler_params=pltpu.CompilerParams(
            fuse_transposed_lhs_in_matmul=True,
            vmem_limit_bytes=100 * 1024 * 1024),
    )(x, memt, jnp.asarray(_EYE))


def _tc_combine(x, b, g, lse):
    return pl.pallas_call(
        _combine_kernel,
        in_specs=[
            pl.BlockSpec((_B, _D), lambda: (0, 0)),
            pl.BlockSpec((_B, _D), lambda: (0, 0)),
            pl.BlockSpec((_NG, 2 * _D), lambda: (0, 0)),
            pl.BlockSpec((_B, 1), lambda: (0, 0)),
        ],
        out_specs=pl.BlockSpec(memory_space=pltpu.SMEM),
        out_shape=jax.ShapeDtypeStruct((1, 1), jnp.float32),
    )(x, b, g, lse)


def kernel(backbone_inputs, inputs, targets, memory_features):
    idx = jnp.concatenate([targets, jnp.repeat(targets[: _B // 4], 4)])
    lse, mem_pad = _tc_lse(inputs, memory_features.T)
    g = _gather_rows(mem_pad, idx)
    loss = _tc_combine(inputs, backbone_inputs, g, lse)
    return loss[0, 0]


# speedup vs baseline: 1.4288x; 1.0040x over previous
import functools
import math

import jax
import jax.numpy as jnp
import numpy as np
from jax import lax
from jax.experimental import pallas as pl
from jax.experimental.pallas import tpu as pltpu
from jax.experimental.pallas import tpu_sc as plsc

_B = 1024
_D = 64
_K = 100000
_TEMP = 0.05
_CHUNK = 8192
_NB = -(-_K // _CHUNK)
_REM = _K - (_NB - 1) * _CHUNK
_NG = 2 * _B
_LOG2E = math.log2(math.e)
_LN2 = math.log(2.0)


def _gather_rows(table, idx):
    info = plsc.get_sparse_core_info()
    nc = info.num_cores
    nw = nc * info.num_subcores
    n = idx.shape[0]
    b_per_w = n // nw

    @functools.partial(
        pl.kernel,
        mesh=plsc.VectorSubcoreMesh(core_axis_name="c", subcore_axis_name="s"),
        out_type=jax.ShapeDtypeStruct((n, 2 * _D), jnp.float32),
        scratch_types=[
            pltpu.VMEM((b_per_w,), jnp.int32),
            pltpu.VMEM((b_per_w, 2 * _D), jnp.float32),
            pltpu.SemaphoreType.DMA,
        ],
    )
    def gath(table_hbm, idx_hbm, out_hbm, idx_v, rows_v, sem):
        wid = lax.axis_index("s") * nc + lax.axis_index("c")
        base = wid * b_per_w
        pltpu.sync_copy(idx_hbm.at[pl.ds(base, b_per_w)], idx_v)
        pltpu.async_copy(table_hbm.at[idx_v], rows_v, sem).wait()
        pltpu.sync_copy(rows_v, out_hbm.at[pl.ds(base, b_per_w)])

    return gath(table, idx)


def _lse_kernel(x_ref, memt_ref, eye_ref, lse_ref, pad_ref,
                xs_ref, m_ref, s_ref):
    i = pl.program_id(0)

    @pl.when(i == 0)
    def _init():
        x = x_ref[...]
        nrm = jnp.sqrt(jnp.sum(x * x, axis=1, keepdims=True))
        xs_ref[...] = x * (_LOG2E / _TEMP) / jnp.maximum(nrm, 1e-12)
        m_ref[...] = jnp.full((_B, 1), -1e30, jnp.float32)
        s_ref[...] = jnp.zeros((_B, 1), jnp.float32)

    memt = memt_ref[...]
    pad_ref[:, 0:_D] = jnp.transpose(memt)
    dot = lax.dot_general(xs_ref[...], memt, (((1,), (0,)), ((), ())),
                          preferred_element_type=jnp.float32)

    def _update(dotv):
        cm = jnp.max(dotv, axis=1, keepdims=True)
        m_old = m_ref[...]
        m_new = jnp.maximum(m_old, cm)
        e = jnp.exp2(dotv - m_new)
        ssum = jnp.sum(e, axis=1, keepdims=True)
        s_ref[...] = s_ref[...] * jnp.exp2(m_old - m_new) + ssum
        m_ref[...] = m_new

    @pl.when(i < _NB - 1)
    def _full():
        _update(dot)

    @pl.when(i == _NB - 1)
    def _partial():
        col = lax.broadcasted_iota(jnp.int32, (_B, _CHUNK), 1)
        _update(jnp.where(col < _REM, dot, -1e30))

    @pl.when(i == _NB - 1)
    def _fin():
        lse_ref[...] = m_ref[...] * _LN2 + jnp.log(s_ref[...])


def _combine_kernel(x_ref, b_ref, g_ref, lse_ref, out_ref):
    x = x_ref[...]
    xn = x / jnp.maximum(
        jnp.sqrt(jnp.sum(x * x, axis=1, keepdims=True)), 1e-12)
    g1 = g_ref[0:_B, 0:_D]
    g2 = g_ref[_B:_NG, 0:_D]
    tdot = jnp.sum(xn * g1, axis=1, keepdims=True) / _TEMP
    loss_ce = jnp.mean(lse_ref[...] - tdot)
    b = b_ref[...]
    bn = b / jnp.maximum(
        jnp.sqrt(jnp.sum(b * b, axis=1, keepdims=True)), 1e-12)
    diff = bn - g2
    loss_d = (0.007 / 0.3) * jnp.sqrt(jnp.sum(diff * diff))
    out_ref[0, 0] = loss_ce + loss_d


_EYE = np.eye(_D, 2 * _D, dtype=np.float32)


def _tc_lse(x, memt):
    return pl.pallas_call(
        _lse_kernel,
        grid=(_NB,),
        in_specs=[
            pl.BlockSpec((_B, _D), lambda i: (0, 0)),
            pl.BlockSpec((_D, _CHUNK), lambda i: (0, i)),
            pl.BlockSpec((_D, 2 * _D), lambda i: (0, 0)),
        ],
        out_specs=[
            pl.BlockSpec((_B, 1), lambda i: (0, 0)),
            pl.BlockSpec((_CHUNK, 2 * _D), lambda i: (i, 0)),
        ],
        out_shape=[
            jax.ShapeDtypeStruct((_B, 1), jnp.float32),
            jax.ShapeDtypeStruct((_K, 2 * _D), jnp.float32),
        ],
        scratch_shapes=[
            pltpu.VMEM((_B, _D), jnp.float32),
            pltpu.VMEM((_B, 1), jnp.float32),
            pltpu.VMEM((_B, 1), jnp.float32),
        ],
        compiler_params=pltpu.CompilerParams(
            fuse_transposed_lhs_in_matmul=True,
            vmem_limit_bytes=100 * 1024 * 1024),
    )(x, memt, jnp.asarray(_EYE))


def _tc_combine(x, b, g, lse):
    return pl.pallas_call(
        _combine_kernel,
        in_specs=[
            pl.BlockSpec((_B, _D), lambda: (0, 0)),
            pl.BlockSpec((_B, _D), lambda: (0, 0)),
            pl.BlockSpec((_NG, 2 * _D), lambda: (0, 0)),
            pl.BlockSpec((_B, 1), lambda: (0, 0)),
        ],
        out_specs=pl.BlockSpec(memory_space=pltpu.SMEM),
        out_shape=jax.ShapeDtypeStruct((1, 1), jnp.float32),
    )(x, b, g, lse)


def kernel(backbone_inputs, inputs, targets, memory_features):
    idx = jnp.concatenate([targets, jnp.repeat(targets[: _B // 4], 4)])
    lse, mem_pad = _tc_lse(inputs, memory_features.T)
    g = _gather_rows(mem_pad, idx)
    loss = _tc_combine(inputs, backbone_inputs, g, lse)
    return loss[0, 0]


# speedup vs baseline: 1.4290x; 1.0002x over previous
import functools
import math

import jax
import jax.numpy as jnp
from jax import lax
from jax.experimental import pallas as pl
from jax.experimental.pallas import tpu as pltpu
from jax.experimental.pallas import tpu_sc as plsc

_B = 1024
_D = 64
_K = 100000
_TEMP = 0.05
_CHUNK = 8192
_NB = -(-_K // _CHUNK)
_REM = _K - (_NB - 1) * _CHUNK
_NG = 2 * _B
_LOG2E = math.log2(math.e)
_LN2 = math.log(2.0)


def _gather_rows(table, idx):
    info = plsc.get_sparse_core_info()
    nc = info.num_cores
    nw = nc * info.num_subcores
    n = idx.shape[0]
    b_per_w = n // nw

    @functools.partial(
        pl.kernel,
        mesh=plsc.VectorSubcoreMesh(core_axis_name="c", subcore_axis_name="s"),
        out_type=jax.ShapeDtypeStruct((n, 2 * _D), jnp.float32),
        scratch_types=[
            pltpu.VMEM((b_per_w,), jnp.int32),
            pltpu.VMEM((b_per_w, 2 * _D), jnp.float32),
            pltpu.SemaphoreType.DMA,
        ],
    )
    def gath(table_hbm, idx_hbm, out_hbm, idx_v, rows_v, sem):
        wid = lax.axis_index("s") * nc + lax.axis_index("c")
        base = wid * b_per_w
        pltpu.sync_copy(idx_hbm.at[pl.ds(base, b_per_w)], idx_v)
        pltpu.async_copy(table_hbm.at[idx_v], rows_v, sem).wait()
        pltpu.sync_copy(rows_v, out_hbm.at[pl.ds(base, b_per_w)])

    return gath(table, idx)


def _lse_kernel(x_ref, memt_ref, lse_ref, pad_ref, xs_ref, m_ref, s_ref):
    i = pl.program_id(0)

    @pl.when(i == 0)
    def _init():
        x = x_ref[...]
        nrm = jnp.sqrt(jnp.sum(x * x, axis=1, keepdims=True))
        xs_ref[...] = x * (_LOG2E / _TEMP) / jnp.maximum(nrm, 1e-12)
        m_ref[...] = jnp.full((_B, 1), -1e30, jnp.float32)
        s_ref[...] = jnp.zeros((_B, 1), jnp.float32)

    memt = memt_ref[...]
    pad_ref[:, 0:_D] = jnp.transpose(memt)
    dot = lax.dot_general(xs_ref[...], memt, (((1,), (0,)), ((), ())),
                          preferred_element_type=jnp.float32)

    def _update(dotv):
        cm = jnp.max(dotv, axis=1, keepdims=True)
        m_old = m_ref[...]
        m_new = jnp.maximum(m_old, cm)
        e = jnp.exp2(dotv - m_new)
        ssum = jnp.sum(e, axis=1, keepdims=True)
        s_ref[...] = s_ref[...] * jnp.exp2(m_old - m_new) + ssum
        m_ref[...] = m_new

    @pl.when(i < _NB - 1)
    def _full():
        _update(dot)

    @pl.when(i == _NB - 1)
    def _partial():
        col = lax.broadcasted_iota(jnp.int32, (_B, _CHUNK), 1)
        _update(jnp.where(col < _REM, dot, -1e30))

    @pl.when(i == _NB - 1)
    def _fin():
        lse_ref[...] = m_ref[...] * _LN2 + jnp.log(s_ref[...])


def _combine_kernel(x_ref, b_ref, g_ref, lse_ref, out_ref):
    x = x_ref[...]
    xn = x / jnp.maximum(
        jnp.sqrt(jnp.sum(x * x, axis=1, keepdims=True)), 1e-12)
    g1 = g_ref[0:_B, 0:_D]
    g2 = g_ref[_B:_NG, 0:_D]
    tdot = jnp.sum(xn * g1, axis=1, keepdims=True) / _TEMP
    loss_ce = jnp.mean(lse_ref[...] - tdot)
    b = b_ref[...]
    bn = b / jnp.maximum(
        jnp.sqrt(jnp.sum(b * b, axis=1, keepdims=True)), 1e-12)
    diff = bn - g2
    loss_d = (0.007 / 0.3) * jnp.sqrt(jnp.sum(diff * diff))
    out_ref[0, 0] = loss_ce + loss_d


def _tc_lse(x, memt):
    return pl.pallas_call(
        _lse_kernel,
        grid=(_NB,),
        in_specs=[
            pl.BlockSpec((_B, _D), lambda i: (0, 0)),
            pl.BlockSpec((_D, _CHUNK), lambda i: (0, i)),
        ],
        out_specs=[
            pl.BlockSpec((_B, 1), lambda i: (0, 0)),
            pl.BlockSpec((_CHUNK, 2 * _D), lambda i: (i, 0)),
        ],
        out_shape=[
            jax.ShapeDtypeStruct((_B, 1), jnp.float32),
            jax.ShapeDtypeStruct((_K, 2 * _D), jnp.float32),
        ],
        scratch_shapes=[
            pltpu.VMEM((_B, _D), jnp.float32),
            pltpu.VMEM((_B, 1), jnp.float32),
            pltpu.VMEM((_B, 1), jnp.float32),
        ],
        compiler_params=pltpu.CompilerParams(
            fuse_transposed_lhs_in_matmul=True,
            vmem_limit_bytes=100 * 1024 * 1024),
    )(x, memt)


def _tc_combine(x, b, g, lse):
    return pl.pallas_call(
        _combine_kernel,
        in_specs=[
            pl.BlockSpec((_B, _D), lambda: (0, 0)),
            pl.BlockSpec((_B, _D), lambda: (0, 0)),
            pl.BlockSpec((_NG, 2 * _D), lambda: (0, 0)),
            pl.BlockSpec((_B, 1), lambda: (0, 0)),
        ],
        out_specs=pl.BlockSpec(memory_space=pltpu.SMEM),
        out_shape=jax.ShapeDtypeStruct((1, 1), jnp.float32),
    )(x, b, g, lse)


def kernel(backbone_inputs, inputs, targets, memory_features):
    idx = jnp.concatenate([targets, jnp.repeat(targets[: _B // 4], 4)])
    lse, mem_pad = _tc_lse(inputs, memory_features.T)
    g = _gather_rows(mem_pad, idx)
    loss = _tc_combine(inputs, backbone_inputs, g, lse)
    return loss[0, 0]
